# Initial kernel scaffold; baseline (speedup 1.0000x reference)
import functools

import jax
import jax.numpy as jnp
from jax import lax
from jax.experimental import pallas as pl
from jax.experimental.pallas import tpu as pltpu
from jax.experimental.pallas import tpu_sc as plsc

N = 10000
D_IN = 128
H1 = 64
H2 = 32

NC = 2
NS = 16
NW = NC * NS
SUB = 128
RPS = 626
N_ACC = NS * RPS

W1 = H1 + 16


def _make_sc_agg(n_sub: int, width: int):
    mesh = plsc.VectorSubcoreMesh(core_axis_name="c", subcore_axis_name="s")

    def body(table, src_i, dst_i, zeros, out, src_v, dst_v, rows_v, acc_sh):
        c = lax.axis_index("c")
        s = lax.axis_index("s")
        wid = c * NS + s

        pltpu.sync_copy(zeros, acc_sh.at[pl.ds(s * RPS, RPS)])
        pltpu.sync_copy(src_i.at[wid], src_v)
        pltpu.sync_copy(dst_i.at[wid], dst_v)
        plsc.subcore_barrier()

        def step(j, carry):
            pltpu.sync_copy(table.at[src_v.at[j]], rows_v)
            pltpu.sync_copy(rows_v, acc_sh.at[dst_v.at[j]], add=True)
            return carry

        lax.fori_loop(0, n_sub, step, 0)
        plsc.subcore_barrier()
        pltpu.sync_copy(acc_sh.at[pl.ds(s * RPS, RPS)],
                        out.at[c, pl.ds(s * RPS, RPS)])

    return pl.kernel(
        body,
        out_type=jax.ShapeDtypeStruct((NC, N_ACC, width), jnp.float32),
        mesh=mesh,
        scratch_types=[
            pltpu.VMEM((n_sub, SUB), jnp.int32),
            pltpu.VMEM((n_sub, SUB), jnp.int32),
            pltpu.VMEM((SUB, width), jnp.float32),
            pltpu.VMEM_SHARED((N_ACC, width), jnp.float32),
        ],
    )


def _proj1_body(x_ref, wl_ref, wr_ref, bl_ref, p_ref, q_ref):
    xb = x_ref[...]
    p = jnp.dot(xb, wl_ref[...], preferred_element_type=jnp.float32)
    ones = jnp.ones((xb.shape[0], W1 - H1), jnp.float32)
    p_ref[...] = jnp.concatenate([p, ones], axis=1)
    q_ref[...] = (
        jnp.dot(xb, wr_ref[...], preferred_element_type=jnp.float32)
        + bl_ref[...]
    )


def _mid_body(a0_ref, a1_ref, q1_ref, wl_ref, wr_ref, bl_ref,
              p2_ref, q2_ref, deg_ref):
    sfull = a0_ref[...] + a1_ref[...]
    deg = jnp.maximum(sfull[:, H1:H1 + 1], 1.0)
    h1 = jax.nn.relu(sfull[:, :H1] / deg + q1_ref[...])
    p2_ref[...] = jnp.dot(h1, wl_ref[...], preferred_element_type=jnp.float32)
    q2_ref[...] = (
        jnp.dot(h1, wr_ref[...], preferred_element_type=jnp.float32)
        + bl_ref[...]
    )
    deg_ref[...] = deg


def _head_body(b0_ref, b1_ref, q2_ref, deg_ref, wa_ref, ba_ref,
               cos_ref, sin_ref):
    sfull = b0_ref[...] + b1_ref[...]
    h2 = jax.nn.relu(sfull / deg_ref[...] + q2_ref[...])
    ang = jnp.dot(h2, wa_ref[...], preferred_element_type=jnp.float32)
    ang = ang + ba_ref[...]
    cos_ref[...] = jnp.cos(ang)
    sin_ref[...] = jnp.sin(ang)


_BLK = 2000


def _full(shape):
    return pl.BlockSpec(shape, lambda i: tuple(0 for _ in shape))


def _rows(width):
    return pl.BlockSpec((_BLK, width), lambda i: (i, 0))


def kernel(x, edge_index, Wl1, bl1, Wr1, Wl2, bl2, Wr2, Wa, ba):
    E = edge_index.shape[1]
    per_batch = NW * SUB
    n_sub = -(-E // per_batch)
    e_pad = n_sub * per_batch
    grid = (N // _BLK,)

    src = edge_index[0].astype(jnp.int32)
    dst = edge_index[1].astype(jnp.int32)
    pad = e_pad - E
    if pad:
        src = jnp.concatenate([src, jnp.zeros((pad,), jnp.int32)])
        dst = jnp.concatenate([dst, jnp.full((pad,), N, jnp.int32)])
    src_i = src.reshape(NW, n_sub, SUB)
    dst_i = dst.reshape(NW, n_sub, SUB)

    p1ext, q1 = pl.pallas_call(
        _proj1_body,
        grid=grid,
        in_specs=[_rows(D_IN), _full((D_IN, H1)), _full((D_IN, H1)),
                  _full((1, H1))],
        out_specs=[_rows(W1), _rows(H1)],
        out_shape=[jax.ShapeDtypeStruct((N, W1), jnp.float32),
                   jax.ShapeDtypeStruct((N, H1), jnp.float32)],
    )(x, Wl1, Wr1, bl1.reshape(1, H1))

    zeros1 = jnp.zeros((RPS, W1), jnp.float32)
    agg1 = _make_sc_agg(n_sub, W1)(p1ext, src_i, dst_i, zeros1)

    p2, q2, deg = pl.pallas_call(
        _mid_body,
        grid=grid,
        in_specs=[_rows(W1), _rows(W1), _rows(H1), _full((H1, H2)),
                  _full((H1, H2)), _full((1, H2))],
        out_specs=[_rows(H2), _rows(H2), _rows(1)],
        out_shape=[jax.ShapeDtypeStruct((N, H2), jnp.float32),
                   jax.ShapeDtypeStruct((N, H2), jnp.float32),
                   jax.ShapeDtypeStruct((N, 1), jnp.float32)],
    )(agg1[0, :N], agg1[1, :N], q1, Wl2, Wr2, bl2.reshape(1, H2))

    zeros2 = jnp.zeros((RPS, H2), jnp.float32)
    agg2 = _make_sc_agg(n_sub, H2)(p2, src_i, dst_i, zeros2)

    cos_t, sin_t = pl.pallas_call(
        _head_body,
        grid=grid,
        in_specs=[_rows(H2), _rows(H2), _rows(H2), _rows(1),
                  _full((H2, 1)), _full((1, 1))],
        out_specs=[_rows(1), _rows(1)],
        out_shape=[jax.ShapeDtypeStruct((N, 1), jnp.float32),
                   jax.ShapeDtypeStruct((N, 1), jnp.float32)],
    )(agg2[0, :N], agg2[1, :N], q2, deg, Wa, ba.reshape(1, 1))

    c = cos_t[:, 0]
    s = sin_t[:, 0]
    row0 = jnp.stack([c, -s], axis=-1)
    row1 = jnp.stack([s, c], axis=-1)
    return jnp.stack([row0, row1], axis=1)


# speedup vs baseline: 6.4081x; 6.4081x over previous
import functools

import jax
import jax.numpy as jnp
from jax import lax
from jax.experimental import pallas as pl
from jax.experimental.pallas import tpu as pltpu
from jax.experimental.pallas import tpu_sc as plsc

N = 10000
D_IN = 128
H1 = 64
H2 = 32

NC = 2
NS = 16
NW = NC * NS
SUB = 128
RPS = 632
N_ACC = NS * RPS

W1 = H1 + 16


def _make_sc_agg(n_sub: int, width: int):
    mesh = plsc.VectorSubcoreMesh(core_axis_name="c", subcore_axis_name="s")

    def body(table, src_i, dst_i, zeros, out, src_v, dst_v, rows_v, acc_sh):
        c = lax.axis_index("c")
        s = lax.axis_index("s")
        wid = c * NS + s

        pltpu.sync_copy(zeros, acc_sh.at[pl.ds(s * RPS, RPS)])
        pltpu.sync_copy(src_i.at[wid], src_v)
        pltpu.sync_copy(dst_i.at[wid], dst_v)
        plsc.subcore_barrier()

        def step(j, carry):
            pltpu.sync_copy(table.at[src_v.at[j]], rows_v)
            pltpu.sync_copy(rows_v, acc_sh.at[dst_v.at[j]], add=True)
            return carry

        lax.fori_loop(0, n_sub, step, 0)
        plsc.subcore_barrier()
        pltpu.sync_copy(acc_sh.at[pl.ds(s * RPS, RPS)],
                        out.at[c, pl.ds(s * RPS, RPS)])

    return pl.kernel(
        body,
        out_type=jax.ShapeDtypeStruct((NC, N_ACC, width), jnp.float32),
        mesh=mesh,
        scratch_types=[
            pltpu.VMEM((n_sub, SUB), jnp.int32),
            pltpu.VMEM((n_sub, SUB), jnp.int32),
            pltpu.VMEM((SUB, width), jnp.float32),
            pltpu.VMEM_SHARED((N_ACC, width), jnp.float32),
        ],
        compiler_params=pltpu.CompilerParams(use_tc_tiling_on_sc=False),
    )


def _proj1_body(x_ref, wl_ref, wr_ref, bl_ref, p_ref, q_ref):
    xb = x_ref[...]
    p = jnp.dot(xb, wl_ref[...], preferred_element_type=jnp.float32)
    ones = jnp.ones((xb.shape[0], W1 - H1), jnp.float32)
    p_ref[...] = jnp.concatenate([p, ones], axis=1)
    q_ref[...] = (
        jnp.dot(xb, wr_ref[...], preferred_element_type=jnp.float32)
        + bl_ref[...]
    )


def _mid_body(a0_ref, a1_ref, q1_ref, wl_ref, wr_ref, bl_ref,
              p2_ref, q2_ref, deg_ref):
    sfull = a0_ref[...] + a1_ref[...]
    deg = jnp.maximum(sfull[:, H1:H1 + 1], 1.0)
    h1 = jax.nn.relu(sfull[:, :H1] / deg + q1_ref[...])
    p2_ref[...] = jnp.dot(h1, wl_ref[...], preferred_element_type=jnp.float32)
    q2_ref[...] = (
        jnp.dot(h1, wr_ref[...], preferred_element_type=jnp.float32)
        + bl_ref[...]
    )
    deg_ref[...] = deg


def _head_body(b0_ref, b1_ref, q2_ref, deg_ref, wa_ref, ba_ref,
               cos_ref, sin_ref):
    sfull = b0_ref[...] + b1_ref[...]
    h2 = jax.nn.relu(sfull / deg_ref[...] + q2_ref[...])
    ang = jnp.dot(h2, wa_ref[...], preferred_element_type=jnp.float32)
    ang = ang + ba_ref[...]
    cos_ref[...] = jnp.cos(ang)
    sin_ref[...] = jnp.sin(ang)


_BLK = 2000


def _full(shape):
    return pl.BlockSpec(shape, lambda i: tuple(0 for _ in shape))


def _rows(width):
    return pl.BlockSpec((_BLK, width), lambda i: (i, 0))


def kernel(x, edge_index, Wl1, bl1, Wr1, Wl2, bl2, Wr2, Wa, ba):
    E = edge_index.shape[1]
    per_batch = NW * SUB
    n_sub = -(-E // per_batch)
    e_pad = n_sub * per_batch
    grid = (N // _BLK,)

    src = edge_index[0].astype(jnp.int32)
    dst = edge_index[1].astype(jnp.int32)
    pad = e_pad - E
    if pad:
        src = jnp.concatenate([src, jnp.zeros((pad,), jnp.int32)])
        dst = jnp.concatenate([dst, jnp.full((pad,), N, jnp.int32)])
    src_i = src.reshape(NW, n_sub, SUB)
    dst_i = dst.reshape(NW, n_sub, SUB)

    p1ext, q1 = pl.pallas_call(
        _proj1_body,
        grid=grid,
        in_specs=[_rows(D_IN), _full((D_IN, H1)), _full((D_IN, H1)),
                  _full((1, H1))],
        out_specs=[_rows(W1), _rows(H1)],
        out_shape=[jax.ShapeDtypeStruct((N, W1), jnp.float32),
                   jax.ShapeDtypeStruct((N, H1), jnp.float32)],
    )(x, Wl1, Wr1, bl1.reshape(1, H1))

    zeros1 = jnp.zeros((RPS, W1), jnp.float32)
    agg1 = _make_sc_agg(n_sub, W1)(p1ext, src_i, dst_i, zeros1)

    p2, q2, deg = pl.pallas_call(
        _mid_body,
        grid=grid,
        in_specs=[_rows(W1), _rows(W1), _rows(H1), _full((H1, H2)),
                  _full((H1, H2)), _full((1, H2))],
        out_specs=[_rows(H2), _rows(H2), _rows(1)],
        out_shape=[jax.ShapeDtypeStruct((N, H2), jnp.float32),
                   jax.ShapeDtypeStruct((N, H2), jnp.float32),
                   jax.ShapeDtypeStruct((N, 1), jnp.float32)],
    )(agg1[0, :N], agg1[1, :N], q1, Wl2, Wr2, bl2.reshape(1, H2))

    zeros2 = jnp.zeros((RPS, H2), jnp.float32)
    agg2 = _make_sc_agg(n_sub, H2)(p2, src_i, dst_i, zeros2)

    cos_t, sin_t = pl.pallas_call(
        _head_body,
        grid=grid,
        in_specs=[_rows(H2), _rows(H2), _rows(H2), _rows(1),
                  _full((H2, 1)), _full((1, 1))],
        out_specs=[_rows(1), _rows(1)],
        out_shape=[jax.ShapeDtypeStruct((N, 1), jnp.float32),
                   jax.ShapeDtypeStruct((N, 1), jnp.float32)],
    )(agg2[0, :N], agg2[1, :N], q2, deg, Wa, ba.reshape(1, 1))

    c = cos_t[:, 0]
    s = sin_t[:, 0]
    row0 = jnp.stack([c, -s], axis=-1)
    row1 = jnp.stack([s, c], axis=-1)
    return jnp.stack([row0, row1], axis=1)


# speedup vs baseline: 8.5723x; 1.3377x over previous
import functools

import jax
import jax.numpy as jnp
from jax import lax
from jax.experimental import pallas as pl
from jax.experimental.pallas import tpu as pltpu
from jax.experimental.pallas import tpu_sc as plsc

N = 10000
D_IN = 128
H1 = 64
H2 = 32

NC = 2
NS = 16
NW = NC * NS
SUB = 128
RPS = 632
N_ACC = NS * RPS

W1 = H1 + 16


NBUF = 4


def _make_sc_agg(n_sub: int, width: int):
    mesh = plsc.VectorSubcoreMesh(core_axis_name="c", subcore_axis_name="s")
    n_outer = -(-n_sub // NBUF)

    def body(table, src_i, dst_i, zeros, out, src_v, dst_v, rows_v, acc_sh,
             sems):
        c = lax.axis_index("c")
        s = lax.axis_index("s")
        wid = c * NS + s

        pltpu.sync_copy(zeros, acc_sh.at[pl.ds(s * RPS, RPS)])
        pltpu.sync_copy(src_i.at[wid], src_v)
        pltpu.sync_copy(dst_i.at[wid], dst_v)
        plsc.subcore_barrier()

        for b in range(NBUF):
            pltpu.async_copy(table.at[src_v.at[b]], rows_v.at[b], sems.at[b])

        def step(o, carry):
            j0 = o * NBUF
            for b in range(NBUF):
                j = j0 + b

                @pl.when(j < n_sub)
                def _():
                    pltpu.make_async_copy(
                        table.at[src_v.at[j]], rows_v.at[b], sems.at[b]
                    ).wait()
                    pltpu.sync_copy(rows_v.at[b], acc_sh.at[dst_v.at[j]],
                                    add=True)

                    @pl.when(j + NBUF < n_sub)
                    def _():
                        pltpu.async_copy(table.at[src_v.at[j + NBUF]],
                                         rows_v.at[b], sems.at[b])
            return carry

        lax.fori_loop(0, n_outer, step, 0)
        plsc.subcore_barrier()
        pltpu.sync_copy(acc_sh.at[pl.ds(s * RPS, RPS)],
                        out.at[c, pl.ds(s * RPS, RPS)])

    return pl.kernel(
        body,
        out_type=jax.ShapeDtypeStruct((NC, N_ACC, width), jnp.float32),
        mesh=mesh,
        scratch_types=[
            pltpu.VMEM((n_sub, SUB), jnp.int32),
            pltpu.VMEM((n_sub, SUB), jnp.int32),
            pltpu.VMEM((NBUF, SUB, width), jnp.float32),
            pltpu.VMEM_SHARED((N_ACC, width), jnp.float32),
            pltpu.SemaphoreType.DMA((NBUF,)),
        ],
        compiler_params=pltpu.CompilerParams(use_tc_tiling_on_sc=False),
    )


def _proj1_body(x_ref, wl_ref, wr_ref, bl_ref, p_ref, q_ref):
    xb = x_ref[...]
    p = jnp.dot(xb, wl_ref[...], preferred_element_type=jnp.float32)
    ones = jnp.ones((xb.shape[0], W1 - H1), jnp.float32)
    p_ref[...] = jnp.concatenate([p, ones], axis=1)
    q_ref[...] = (
        jnp.dot(xb, wr_ref[...], preferred_element_type=jnp.float32)
        + bl_ref[...]
    )


def _mid_body(a0_ref, a1_ref, q1_ref, wl_ref, wr_ref, bl_ref,
              p2_ref, q2_ref, deg_ref):
    sfull = a0_ref[...] + a1_ref[...]
    deg = jnp.maximum(sfull[:, H1:H1 + 1], 1.0)
    h1 = jax.nn.relu(sfull[:, :H1] / deg + q1_ref[...])
    p2_ref[...] = jnp.dot(h1, wl_ref[...], preferred_element_type=jnp.float32)
    q2_ref[...] = (
        jnp.dot(h1, wr_ref[...], preferred_element_type=jnp.float32)
        + bl_ref[...]
    )
    deg_ref[...] = deg


def _head_body(b0_ref, b1_ref, q2_ref, deg_ref, wa_ref, ba_ref,
               cos_ref, sin_ref):
    sfull = b0_ref[...] + b1_ref[...]
    h2 = jax.nn.relu(sfull / deg_ref[...] + q2_ref[...])
    ang = jnp.dot(h2, wa_ref[...], preferred_element_type=jnp.float32)
    ang = ang + ba_ref[...]
    cos_ref[...] = jnp.cos(ang)
    sin_ref[...] = jnp.sin(ang)


_BLK = 2000


def _full(shape):
    return pl.BlockSpec(shape, lambda i: tuple(0 for _ in shape))


def _rows(width):
    return pl.BlockSpec((_BLK, width), lambda i: (i, 0))


def kernel(x, edge_index, Wl1, bl1, Wr1, Wl2, bl2, Wr2, Wa, ba):
    E = edge_index.shape[1]
    per_batch = NW * SUB
    n_sub = -(-E // per_batch)
    e_pad = n_sub * per_batch
    grid = (N // _BLK,)

    src = edge_index[0].astype(jnp.int32)
    dst = edge_index[1].astype(jnp.int32)
    pad = e_pad - E
    if pad:
        src = jnp.concatenate([src, jnp.zeros((pad,), jnp.int32)])
        dst = jnp.concatenate([dst, jnp.full((pad,), N, jnp.int32)])
    src_i = src.reshape(NW, n_sub, SUB)
    dst_i = dst.reshape(NW, n_sub, SUB)

    p1ext, q1 = pl.pallas_call(
        _proj1_body,
        grid=grid,
        in_specs=[_rows(D_IN), _full((D_IN, H1)), _full((D_IN, H1)),
                  _full((1, H1))],
        out_specs=[_rows(W1), _rows(H1)],
        out_shape=[jax.ShapeDtypeStruct((N, W1), jnp.float32),
                   jax.ShapeDtypeStruct((N, H1), jnp.float32)],
    )(x, Wl1, Wr1, bl1.reshape(1, H1))

    zeros1 = jnp.zeros((RPS, W1), jnp.float32)
    agg1 = _make_sc_agg(n_sub, W1)(p1ext, src_i, dst_i, zeros1)

    p2, q2, deg = pl.pallas_call(
        _mid_body,
        grid=grid,
        in_specs=[_rows(W1), _rows(W1), _rows(H1), _full((H1, H2)),
                  _full((H1, H2)), _full((1, H2))],
        out_specs=[_rows(H2), _rows(H2), _rows(1)],
        out_shape=[jax.ShapeDtypeStruct((N, H2), jnp.float32),
                   jax.ShapeDtypeStruct((N, H2), jnp.float32),
                   jax.ShapeDtypeStruct((N, 1), jnp.float32)],
    )(agg1[0, :N], agg1[1, :N], q1, Wl2, Wr2, bl2.reshape(1, H2))

    zeros2 = jnp.zeros((RPS, H2), jnp.float32)
    agg2 = _make_sc_agg(n_sub, H2)(p2, src_i, dst_i, zeros2)

    cos_t, sin_t = pl.pallas_call(
        _head_body,
        grid=grid,
        in_specs=[_rows(H2), _rows(H2), _rows(H2), _rows(1),
                  _full((H2, 1)), _full((1, 1))],
        out_specs=[_rows(1), _rows(1)],
        out_shape=[jax.ShapeDtypeStruct((N, 1), jnp.float32),
                   jax.ShapeDtypeStruct((N, 1), jnp.float32)],
    )(agg2[0, :N], agg2[1, :N], q2, deg, Wa, ba.reshape(1, 1))

    c = cos_t[:, 0]
    s = sin_t[:, 0]
    row0 = jnp.stack([c, -s], axis=-1)
    row1 = jnp.stack([s, c], axis=-1)
    return jnp.stack([row0, row1], axis=1)
